# Initial kernel scaffold; baseline (speedup 1.0000x reference)
import jax
import jax.numpy as jnp
from jax.experimental import pallas as pl


def kernel(x, edges, node_mask, edge_mask, temb, W_lin, b_lin, W_lin1, W_temb, b_temb, W_att1, b_att1, W_att2, b_att2, W_em1, b_em1, W_em2, b_em2, W_nm1, b_nm1, W_nm2, b_nm2):
    raise NotImplementedError("write your pallas kernel here")


# speedup vs baseline: 2.6122x; 2.6122x over previous
import functools

import jax
import jax.numpy as jnp
from jax import lax
from jax.experimental import pallas as pl
from jax.experimental.pallas import tpu as pltpu
from jax.experimental.pallas import tpu_sc as plsc

_N = 10000
_E = 320000
_D = 128

_NC = 2
_NS = 16
_NW = _NC * _NS
_EPW = _E // _NW
_CH = 80
_NCH = _EPW // _CH
_NPAD = 10240
_RPT = _NPAD // _NS

_BN = 1000
_BE = 1280


def _silu(v):
    return v * jax.nn.sigmoid(v)


def _pre_body(x_ref, temb_ref, wl_ref, bl_ref, wt_ref, bt_ref, wl1_ref, h_ref):
    t = _silu(temb_ref[...]) @ wt_ref[...]
    u = x_ref[...] @ wl_ref[...] + bl_ref[...] + t + bt_ref[...]
    h_ref[...] = u @ wl1_ref[...]


def _preproc(x, temb, W_lin, b_lin, W_temb, b_temb, W_lin1):
    row = pl.BlockSpec((_BN, _D), lambda i: (i, 0))
    full = pl.BlockSpec((_D, _D), lambda i: (0, 0))
    vec = pl.BlockSpec((1, _D), lambda i: (0, 0))
    return pl.pallas_call(
        _pre_body,
        grid=(_N // _BN,),
        in_specs=[row, row, full, vec, full, vec, full],
        out_specs=row,
        out_shape=jax.ShapeDtypeStruct((_N, _D), jnp.float32),
    )(x, temb, W_lin, b_lin.reshape(1, _D), W_temb, b_temb.reshape(1, _D), W_lin1)


def _gather_body(h_hbm, ridx_hbm, cidx_hbm, hr_hbm, hc_hbm,
                 idxr, idxc, bufr, bufc, semr, semc):
    w = lax.axis_index("s") * _NC + lax.axis_index("c")

    def body(j, carry):
        base = w * _EPW + j * _CH
        pltpu.sync_copy(ridx_hbm.at[pl.ds(base, _CH)], idxr)
        pltpu.sync_copy(cidx_hbm.at[pl.ds(base, _CH)], idxc)
        cr = pltpu.async_copy(h_hbm.at[idxr], bufr, semr)
        cc = pltpu.async_copy(h_hbm.at[idxc], bufc, semc)
        cr.wait()
        cc.wait()
        pltpu.sync_copy(bufr, hr_hbm.at[pl.ds(base, _CH)])
        pltpu.sync_copy(bufc, hc_hbm.at[pl.ds(base, _CH)])
        return carry

    lax.fori_loop(0, _NCH, body, 0)


_gather = functools.partial(
    pl.kernel,
    mesh=plsc.VectorSubcoreMesh(core_axis_name="c", subcore_axis_name="s"),
    out_type=[jax.ShapeDtypeStruct((_E, _D), jnp.float32),
              jax.ShapeDtypeStruct((_E, _D), jnp.float32)],
    scratch_types=[pltpu.VMEM((_CH,), jnp.int32),
                   pltpu.VMEM((_CH,), jnp.int32),
                   pltpu.VMEM((_CH, _D), jnp.float32),
                   pltpu.VMEM((_CH, _D), jnp.float32),
                   pltpu.SemaphoreType.DMA,
                   pltpu.SemaphoreType.DMA],
)(_gather_body)


def _edge_body(hr_ref, hc_ref, em_ref, war, wac, ba1, wa2, ba2,
               wer, wec, be1, we2, be2, m_ref):
    hr = hr_ref[...]
    hc = hc_ref[...]
    pa = hr @ war[...] + hc @ wac[...] + ba1[...]
    t = _silu(pa) @ wa2[...] + ba2[...]
    att = jax.nn.sigmoid(t) * em_ref[...]
    pe = hr @ wer[...] + hc @ wec[...] + be1[...]
    m = _silu(_silu(pe) @ we2[...] + be2[...])
    m_ref[...] = m * att


def _edge_mlp(hr, hc, edge_mask, W_att1, b_att1, W_att2, b_att2,
              W_em1, b_em1, W_em2, b_em2):
    row = pl.BlockSpec((_BE, _D), lambda i: (i, 0))
    mask = pl.BlockSpec((_BE, 1), lambda i: (i, 0))
    full = pl.BlockSpec((_D, _D), lambda i: (0, 0))
    vec = pl.BlockSpec((1, _D), lambda i: (0, 0))
    w2 = pl.BlockSpec((_D, 1), lambda i: (0, 0))
    s2 = pl.BlockSpec((1, 1), lambda i: (0, 0))
    return pl.pallas_call(
        _edge_body,
        grid=(_E // _BE,),
        in_specs=[row, row, mask, full, full, vec, w2, s2,
                  full, full, vec, full, vec],
        out_specs=row,
        out_shape=jax.ShapeDtypeStruct((_E, _D), jnp.float32),
    )(hr, hc, edge_mask,
      W_att1[:_D], W_att1[_D:], b_att1.reshape(1, _D),
      W_att2, b_att2.reshape(1, 1),
      W_em1[:_D], W_em1[_D:], b_em1.reshape(1, _D),
      W_em2, b_em2.reshape(1, _D))


def _scatter_body(m_hbm, ridx_hbm, z_hbm, out_hbm, idxv, mbuf, aggsh):
    c = lax.axis_index("c")
    s = lax.axis_index("s")
    w = s * _NC + c
    pltpu.sync_copy(z_hbm.at[pl.ds(s * _RPT, _RPT)],
                    aggsh.at[pl.ds(s * _RPT, _RPT)])
    plsc.subcore_barrier()

    def body(j, carry):
        base = w * _EPW + j * _CH
        pltpu.sync_copy(ridx_hbm.at[pl.ds(base, _CH)], idxv)
        pltpu.sync_copy(m_hbm.at[pl.ds(base, _CH)], mbuf)
        pltpu.sync_copy(mbuf, aggsh.at[idxv], add=True)
        return carry

    lax.fori_loop(0, _NCH, body, 0)
    plsc.subcore_barrier()
    pltpu.sync_copy(aggsh.at[pl.ds(s * _RPT, _RPT)],
                    out_hbm.at[c, pl.ds(s * _RPT, _RPT)])


_scatter = functools.partial(
    pl.kernel,
    mesh=plsc.VectorSubcoreMesh(core_axis_name="c", subcore_axis_name="s"),
    out_type=jax.ShapeDtypeStruct((_NC, _NPAD, _D), jnp.float32),
    scratch_types=[pltpu.VMEM((_CH,), jnp.int32),
                   pltpu.VMEM((_CH, _D), jnp.float32),
                   pltpu.VMEM_SHARED((_NPAD, _D), jnp.float32)],
)(_scatter_body)


def _node_body(h_ref, p0_ref, p1_ref, nm_ref, wnr, wna, bn1, wn2, bn2, o_ref):
    h = h_ref[...]
    agg = p0_ref[...] + p1_ref[...]
    u = _silu(h @ wnr[...] + agg @ wna[...] + bn1[...])
    o_ref[...] = (h + u @ wn2[...] + bn2[...]) * nm_ref[...]


def _node_mlp(h, p0, p1, node_mask, W_nm1, b_nm1, W_nm2, b_nm2):
    row = pl.BlockSpec((_BN, _D), lambda i: (i, 0))
    mask = pl.BlockSpec((_BN, 1), lambda i: (i, 0))
    full = pl.BlockSpec((_D, _D), lambda i: (0, 0))
    vec = pl.BlockSpec((1, _D), lambda i: (0, 0))
    return pl.pallas_call(
        _node_body,
        grid=(_N // _BN,),
        in_specs=[row, row, row, mask, full, full, vec, full, vec],
        out_specs=row,
        out_shape=jax.ShapeDtypeStruct((_N, _D), jnp.float32),
    )(h, p0, p1, node_mask,
      W_nm1[:_D], W_nm1[_D:], b_nm1.reshape(1, _D), W_nm2, b_nm2.reshape(1, _D))


def kernel(x, edges, node_mask, edge_mask, temb,
           W_lin, b_lin, W_lin1, W_temb, b_temb,
           W_att1, b_att1, W_att2, b_att2,
           W_em1, b_em1, W_em2, b_em2,
           W_nm1, b_nm1, W_nm2, b_nm2):
    ridx = edges[0]
    cidx = edges[1]

    h = _preproc(x, temb, W_lin, b_lin, W_temb, b_temb, W_lin1)
    hr, hc = _gather(h, ridx, cidx)
    m = _edge_mlp(hr, hc, edge_mask, W_att1, b_att1, W_att2, b_att2,
                  W_em1, b_em1, W_em2, b_em2)
    zeros = jnp.zeros((_NPAD, _D), jnp.float32)
    part = _scatter(m, ridx, zeros)
    out = _node_mlp(h, part[0], part[1], node_mask, W_nm1, b_nm1, W_nm2, b_nm2)
    return out
